# baseline (speedup 1.0000x reference)
import functools

import jax
import jax.numpy as jnp
from jax import lax
from jax.experimental import pallas as pl
from jax.experimental.pallas import tpu as pltpu
from jax.experimental.pallas import tpu_sc as plsc

NN = 10000
DD = 128
HH = 256
EE = 320000
EPSM = 1e-7
SHIFT = 30.0

NCORE = 2
NSUB = 16
EPT = EE // NSUB
CHUNK = 72
NCHUNK = 288
BIDX = 24
NIDX = NCHUNK // BIDX
NBUF = 3
DW = DD // 2
EPTP = CHUNK * NCHUNK
NP = 10112
RPT = NP // NSUB
PADROW = NN

BLK = 2000
NBLK = NN // BLK


def _sc_edge_body(t0_hbm, t1_hbm, src_hbm, dst_hbm, zeros_hbm, out_hbm,
                  acc, srcs_v, dsts_v, rows_i, rows_f, gsems, ssems):
    c = lax.axis_index("c")
    s = lax.axis_index("s")
    stripe = pl.ds(s * RPT, RPT)

    pltpu.sync_copy(zeros_hbm, acc.at[stripe])
    plsc.subcore_barrier()

    def expand(src_v, dst_v):
        for r in range(CHUNK):
            for g in range(4):
                w = src_v[r, pl.ds(16 * g, 16)]
                lo = lax.bitcast_convert_type(w << 16, jnp.float32)
                hi = lax.bitcast_convert_type(w & jnp.int32(-65536),
                                              jnp.float32)
                dst_v[r, pl.ds(32 * g, 16)] = lo
                dst_v[r, pl.ds(32 * g + 16, 16)] = hi

    def fire_gather(j, b):
        @pl.when((c == 0) & (j < BIDX))
        def _():
            pltpu.async_copy(t0_hbm.at[srcs_v.at[j]], rows_i[b], gsems[b])

        @pl.when((c == 1) & (j < BIDX))
        def _():
            pltpu.async_copy(t1_hbm.at[srcs_v.at[j]], rows_i[b], gsems[b])

    def wait_gather(b):
        pltpu.make_async_copy(t0_hbm.at[srcs_v.at[0]], rows_i[b],
                              gsems[b]).wait()

    def wait_scatter(b):
        pltpu.make_async_copy(zeros_hbm.at[pl.ds(0, CHUNK)], rows_f[b],
                              ssems[b]).wait()

    def unit(j, b):
        nb = (b + 2) % NBUF
        wait_gather(b)
        expand(rows_i[b], rows_f[b])
        pltpu.async_copy(rows_f[b], acc.at[dsts_v.at[j]], ssems[b],
                         add=True)

        @pl.when((j >= 1) & (j + 2 < BIDX))
        def _():
            wait_scatter(nb)

        fire_gather(j + 2, nb)

    def body(k, carry):
        for u in range(NBUF):
            unit(NBUF * k + u, u)
        return carry

    def block(blkid, carry):
        blk = pl.ds(blkid * BIDX, BIDX)
        pltpu.sync_copy(src_hbm.at[s, blk], srcs_v)
        pltpu.sync_copy(dst_hbm.at[s, blk], dsts_v)
        fire_gather(0, 0)
        fire_gather(1, 1)
        lax.fori_loop(0, BIDX // NBUF, body, 0)
        for b in range(NBUF):
            wait_scatter(b)
        return carry

    lax.fori_loop(0, NIDX, block, 0)

    plsc.subcore_barrier()

    @pl.when(c == 0)
    def _():
        pltpu.sync_copy(acc.at[stripe], out_hbm.at[0, stripe])

    @pl.when(c == 1)
    def _():
        pltpu.sync_copy(acc.at[stripe], out_hbm.at[1, stripe])


@functools.cache
def _sc_edge():
    return pl.kernel(
        _sc_edge_body,
        out_type=jax.ShapeDtypeStruct((NCORE, NP, DD), jnp.float32),
        mesh=plsc.VectorSubcoreMesh(core_axis_name="c", subcore_axis_name="s",
                                    num_cores=NCORE, num_subcores=NSUB),
        compiler_params=pltpu.CompilerParams(use_tc_tiling_on_sc=False),
        scratch_types=[
            pltpu.VMEM_SHARED((NP, DD), jnp.float32),
            pltpu.VMEM((BIDX, CHUNK), jnp.int32),
            pltpu.VMEM((BIDX, CHUNK), jnp.int32),
            [pltpu.VMEM((CHUNK, DW), jnp.int32)] * NBUF,
            [pltpu.VMEM((CHUNK, DD), jnp.float32)] * NBUF,
            [pltpu.SemaphoreType.DMA] * NBUF,
            [pltpu.SemaphoreType.DMA] * NBUF,
        ],
    )


def _tables_from(y):
    m = jnp.maximum(y, 0.0) + EPSM
    p = jnp.exp(m - SHIFT)
    q = m * p
    t0 = jnp.concatenate([q[:, :DD // 2], p[:, :DD // 2]], axis=1)
    t1 = jnp.concatenate([q[:, DD // 2:], p[:, DD // 2:]], axis=1)
    return jnp.stack([t0, t1], axis=0)


def _prep_body(x_ref, t_ref):
    t_ref[...] = _tables_from(x_ref[...])


_prep = pl.pallas_call(
    _prep_body,
    grid=(NBLK,),
    in_specs=[pl.BlockSpec((BLK, DD), lambda i: (i, 0))],
    out_specs=pl.BlockSpec((NCORE, BLK, DD), lambda i: (0, i, 0)),
    out_shape=jax.ShapeDtypeStruct((NCORE, NN, DD), jnp.float32),
)


def _mlp1_body(acc_ref, x_ref, w1_ref, b1_ref, h1_ref, st_ref):
    num = jnp.concatenate([acc_ref[0, :, :DD // 2], acc_ref[1, :, :DD // 2]],
                          axis=1)
    den = jnp.concatenate([acc_ref[0, :, DD // 2:], acc_ref[1, :, DD // 2:]],
                          axis=1)
    agg = num / (den + 1e-16)
    h = agg + x_ref[...]
    h1 = jnp.dot(h, w1_ref[...], preferred_element_type=jnp.float32)
    h1 = h1 + b1_ref[...]
    h1_ref[...] = h1
    blk = jnp.concatenate([jnp.sum(h1, axis=0, keepdims=True),
                           jnp.sum(h1 * h1, axis=0, keepdims=True)], axis=0)

    @pl.when(pl.program_id(0) == 0)
    def _():
        st_ref[...] = blk

    @pl.when(pl.program_id(0) != 0)
    def _():
        st_ref[...] += blk


_mlp1 = pl.pallas_call(
    _mlp1_body,
    grid=(NBLK,),
    in_specs=[
        pl.BlockSpec((NCORE, BLK, DD), lambda i: (0, i, 0)),
        pl.BlockSpec((BLK, DD), lambda i: (i, 0)),
        pl.BlockSpec((DD, HH), lambda i: (0, 0)),
        pl.BlockSpec((1, HH), lambda i: (0, 0)),
    ],
    out_specs=[
        pl.BlockSpec((BLK, HH), lambda i: (i, 0)),
        pl.BlockSpec((2, HH), lambda i: (0, 0)),
    ],
    out_shape=[
        jax.ShapeDtypeStruct((NN, HH), jnp.float32),
        jax.ShapeDtypeStruct((2, HH), jnp.float32),
    ],
)


def _mlp2_body(h1_ref, st_ref, g_ref, be_ref, w2_ref, b2_ref, y_ref, t_ref):
    mean = st_ref[0:1, :] * (1.0 / NN)
    ex2 = st_ref[1:2, :] * (1.0 / NN)
    var = ex2 - mean * mean
    h1n = g_ref[...] * (h1_ref[...] - mean) * lax.rsqrt(var + 1e-5) + be_ref[...]
    r = jnp.maximum(h1n, 0.0)
    y = jnp.dot(r, w2_ref[...], preferred_element_type=jnp.float32)
    y = y + b2_ref[...]
    y = jnp.where(y > 0.0, y, jnp.exp(jnp.minimum(y, 0.0)) - 1.0)
    y_ref[...] = y
    t_ref[...] = _tables_from(y)


_mlp2 = pl.pallas_call(
    _mlp2_body,
    grid=(NBLK,),
    in_specs=[
        pl.BlockSpec((BLK, HH), lambda i: (i, 0)),
        pl.BlockSpec((2, HH), lambda i: (0, 0)),
        pl.BlockSpec((1, HH), lambda i: (0, 0)),
        pl.BlockSpec((1, HH), lambda i: (0, 0)),
        pl.BlockSpec((HH, DD), lambda i: (0, 0)),
        pl.BlockSpec((1, DD), lambda i: (0, 0)),
    ],
    out_specs=[
        pl.BlockSpec((BLK, DD), lambda i: (i, 0)),
        pl.BlockSpec((NCORE, BLK, DD), lambda i: (0, i, 0)),
    ],
    out_shape=[
        jax.ShapeDtypeStruct((NN, DD), jnp.float32),
        jax.ShapeDtypeStruct((NCORE, NN, DD), jnp.float32),
    ],
)


def _pack_tables(t):
    tb = t.astype(jnp.bfloat16).reshape(NCORE, NN, 4, 2, 16).swapaxes(3, 4)
    return jax.lax.bitcast_convert_type(tb, jnp.int32).reshape(NCORE, NN, DW)


def kernel(x, edge_index, W1_0, b1_0, gamma_0, beta_0, W2_0, b2_0,
           W1_1, b1_1, gamma_1, beta_1, W2_1, b2_1):
    def tile_ids(flat, pad_value):
        per_tile = flat.astype(jnp.int32).reshape(NSUB, EPT)
        pad = jnp.full((NSUB, EPTP - EPT), pad_value, jnp.int32)
        return jnp.concatenate([per_tile, pad], axis=1).reshape(
            NSUB, NCHUNK, CHUNK)

    src = tile_ids(edge_index[0], 0)
    dst = tile_ids(edge_index[1], PADROW)
    zeros = jnp.zeros((RPT, DD), jnp.float32)

    def layer(tables, xin, W1, b1, g, be, W2, b2):
        packed = _pack_tables(tables)
        acc = _sc_edge()(packed[0], packed[1], src, dst, zeros)[:, :NN, :]
        h1, st = _mlp1(acc, xin, W1, b1.reshape(1, HH))
        y, tnext = _mlp2(h1, st, g.reshape(1, HH), be.reshape(1, HH),
                         W2, b2.reshape(1, DD))
        return y, tnext

    t = _prep(x)
    y0, t = layer(t, x, W1_0, b1_0, gamma_0, beta_0, W2_0, b2_0)
    y1, _ = layer(t, y0, W1_1, b1_1, gamma_1, beta_1, W2_1, b2_1)
    return y1


# speedup vs baseline: 1.3560x; 1.3560x over previous
import functools

import jax
import jax.numpy as jnp
from jax import lax
from jax.experimental import pallas as pl
from jax.experimental.pallas import tpu as pltpu
from jax.experimental.pallas import tpu_sc as plsc

NN = 10000
DD = 128
HH = 256
EE = 320000
EPSM = 1e-7
SHIFT = 30.0

NCORE = 2
NSUB = 16
EPT = EE // NSUB
CHUNK = 80
NCHUNK = 256
BIDX = 64
NIDX = NCHUNK // BIDX
NBUF = 2
DW = DD // 2
EPTP = CHUNK * NCHUNK
NP = 10112
RPT = NP // NSUB
PADROW = NN

BLK = 2000
NBLK = NN // BLK


def _sc_edge_body(t0_hbm, t1_hbm, src_hbm, dst_hbm, zeros_hbm, out_hbm,
                  acc, srcs_v, dsts_v, rows_i, rows_f, gsems, ssems):
    c = lax.axis_index("c")
    s = lax.axis_index("s")
    stripe = pl.ds(s * RPT, RPT)

    pltpu.sync_copy(zeros_hbm, acc.at[stripe])
    plsc.subcore_barrier()

    def expand(src_v, dst_v):
        for r in range(CHUNK):
            for g in range(4):
                w = src_v[r, pl.ds(16 * g, 16)]
                lo = lax.bitcast_convert_type(w << 16, jnp.float32)
                hi = lax.bitcast_convert_type(w & jnp.int32(-65536),
                                              jnp.float32)
                dst_v[r, pl.ds(32 * g, 16)] = lo
                dst_v[r, pl.ds(32 * g + 16, 16)] = hi

    def fire_gather(j, b):
        @pl.when((c == 0) & (j < BIDX))
        def _():
            pltpu.async_copy(t0_hbm.at[srcs_v.at[j]], rows_i[b], gsems[b])

        @pl.when((c == 1) & (j < BIDX))
        def _():
            pltpu.async_copy(t1_hbm.at[srcs_v.at[j]], rows_i[b], gsems[b])

    def wait_gather(b):
        pltpu.make_async_copy(t0_hbm.at[srcs_v.at[0]], rows_i[b],
                              gsems[b]).wait()

    def wait_scatter(b):
        pltpu.make_async_copy(zeros_hbm.at[pl.ds(0, CHUNK)], rows_f[b],
                              ssems[b]).wait()

    def unit(j, b):
        wait_gather(b)

        @pl.when(j >= 2)
        def _():
            wait_scatter(b)

        expand(rows_i[b], rows_f[b])
        pltpu.async_copy(rows_f[b], acc.at[dsts_v.at[j]], ssems[b],
                         add=True)
        fire_gather(j + 2, b)

    def body(k, carry):
        for u in range(NBUF):
            unit(NBUF * k + u, u)
        return carry

    def block(blkid, carry):
        blk = pl.ds(blkid * BIDX, BIDX)
        pltpu.sync_copy(src_hbm.at[s, blk], srcs_v)
        pltpu.sync_copy(dst_hbm.at[s, blk], dsts_v)
        fire_gather(0, 0)
        fire_gather(1, 1)
        lax.fori_loop(0, BIDX // NBUF, body, 0)
        for b in range(NBUF):
            wait_scatter(b)
        return carry

    lax.fori_loop(0, NIDX, block, 0)

    plsc.subcore_barrier()

    @pl.when(c == 0)
    def _():
        pltpu.sync_copy(acc.at[stripe], out_hbm.at[0, stripe])

    @pl.when(c == 1)
    def _():
        pltpu.sync_copy(acc.at[stripe], out_hbm.at[1, stripe])


@functools.cache
def _sc_edge():
    return pl.kernel(
        _sc_edge_body,
        out_type=jax.ShapeDtypeStruct((NCORE, NP, DD), jnp.float32),
        mesh=plsc.VectorSubcoreMesh(core_axis_name="c", subcore_axis_name="s",
                                    num_cores=NCORE, num_subcores=NSUB),
        compiler_params=pltpu.CompilerParams(use_tc_tiling_on_sc=False),
        scratch_types=[
            pltpu.VMEM_SHARED((NP, DD), jnp.float32),
            pltpu.VMEM((BIDX, CHUNK), jnp.int32),
            pltpu.VMEM((BIDX, CHUNK), jnp.int32),
            [pltpu.VMEM((CHUNK, DW), jnp.int32)] * NBUF,
            [pltpu.VMEM((CHUNK, DD), jnp.float32)] * NBUF,
            [pltpu.SemaphoreType.DMA] * NBUF,
            [pltpu.SemaphoreType.DMA] * NBUF,
        ],
    )


def _tables_from(y):
    m = jnp.maximum(y, 0.0) + EPSM
    p = jnp.exp(m - SHIFT)
    q = m * p
    t0 = jnp.concatenate([q[:, :DD // 2], p[:, :DD // 2]], axis=1)
    t1 = jnp.concatenate([q[:, DD // 2:], p[:, DD // 2:]], axis=1)
    return jnp.stack([t0, t1], axis=0)


def _prep_body(x_ref, t_ref):
    t_ref[...] = _tables_from(x_ref[...])


_prep = pl.pallas_call(
    _prep_body,
    grid=(NBLK,),
    in_specs=[pl.BlockSpec((BLK, DD), lambda i: (i, 0))],
    out_specs=pl.BlockSpec((NCORE, BLK, DD), lambda i: (0, i, 0)),
    out_shape=jax.ShapeDtypeStruct((NCORE, NN, DD), jnp.float32),
)


def _mlp1_body(acc_ref, x_ref, w1_ref, b1_ref, h1_ref, st_ref):
    num = jnp.concatenate([acc_ref[0, :, :DD // 2], acc_ref[1, :, :DD // 2]],
                          axis=1)
    den = jnp.concatenate([acc_ref[0, :, DD // 2:], acc_ref[1, :, DD // 2:]],
                          axis=1)
    agg = num / (den + 1e-16)
    h = agg + x_ref[...]
    h1 = jnp.dot(h, w1_ref[...], preferred_element_type=jnp.float32)
    h1 = h1 + b1_ref[...]
    h1_ref[...] = h1
    blk = jnp.concatenate([jnp.sum(h1, axis=0, keepdims=True),
                           jnp.sum(h1 * h1, axis=0, keepdims=True)], axis=0)

    @pl.when(pl.program_id(0) == 0)
    def _():
        st_ref[...] = blk

    @pl.when(pl.program_id(0) != 0)
    def _():
        st_ref[...] += blk


_mlp1 = pl.pallas_call(
    _mlp1_body,
    grid=(NBLK,),
    in_specs=[
        pl.BlockSpec((NCORE, BLK, DD), lambda i: (0, i, 0)),
        pl.BlockSpec((BLK, DD), lambda i: (i, 0)),
        pl.BlockSpec((DD, HH), lambda i: (0, 0)),
        pl.BlockSpec((1, HH), lambda i: (0, 0)),
    ],
    out_specs=[
        pl.BlockSpec((BLK, HH), lambda i: (i, 0)),
        pl.BlockSpec((2, HH), lambda i: (0, 0)),
    ],
    out_shape=[
        jax.ShapeDtypeStruct((NN, HH), jnp.float32),
        jax.ShapeDtypeStruct((2, HH), jnp.float32),
    ],
)


def _mlp2_body(h1_ref, st_ref, g_ref, be_ref, w2_ref, b2_ref, y_ref, t_ref):
    mean = st_ref[0:1, :] * (1.0 / NN)
    ex2 = st_ref[1:2, :] * (1.0 / NN)
    var = ex2 - mean * mean
    h1n = g_ref[...] * (h1_ref[...] - mean) * lax.rsqrt(var + 1e-5) + be_ref[...]
    r = jnp.maximum(h1n, 0.0)
    y = jnp.dot(r, w2_ref[...], preferred_element_type=jnp.float32)
    y = y + b2_ref[...]
    y = jnp.where(y > 0.0, y, jnp.exp(jnp.minimum(y, 0.0)) - 1.0)
    y_ref[...] = y
    t_ref[...] = _tables_from(y)


_mlp2 = pl.pallas_call(
    _mlp2_body,
    grid=(NBLK,),
    in_specs=[
        pl.BlockSpec((BLK, HH), lambda i: (i, 0)),
        pl.BlockSpec((2, HH), lambda i: (0, 0)),
        pl.BlockSpec((1, HH), lambda i: (0, 0)),
        pl.BlockSpec((1, HH), lambda i: (0, 0)),
        pl.BlockSpec((HH, DD), lambda i: (0, 0)),
        pl.BlockSpec((1, DD), lambda i: (0, 0)),
    ],
    out_specs=[
        pl.BlockSpec((BLK, DD), lambda i: (i, 0)),
        pl.BlockSpec((NCORE, BLK, DD), lambda i: (0, i, 0)),
    ],
    out_shape=[
        jax.ShapeDtypeStruct((NN, DD), jnp.float32),
        jax.ShapeDtypeStruct((NCORE, NN, DD), jnp.float32),
    ],
)


def _pack_tables(t):
    tb = t.astype(jnp.bfloat16).reshape(NCORE, NN, 4, 2, 16).swapaxes(3, 4)
    return jax.lax.bitcast_convert_type(tb, jnp.int32).reshape(NCORE, NN, DW)


def kernel(x, edge_index, W1_0, b1_0, gamma_0, beta_0, W2_0, b2_0,
           W1_1, b1_1, gamma_1, beta_1, W2_1, b2_1):
    def tile_ids(flat, pad_value):
        per_tile = flat.astype(jnp.int32).reshape(NSUB, EPT)
        pad = jnp.full((NSUB, EPTP - EPT), pad_value, jnp.int32)
        return jnp.concatenate([per_tile, pad], axis=1).reshape(
            NSUB, NCHUNK, CHUNK)

    src = tile_ids(edge_index[0], 0)
    dst = tile_ids(edge_index[1], PADROW)
    zeros = jnp.zeros((RPT, DD), jnp.float32)

    def layer(tables, xin, W1, b1, g, be, W2, b2):
        packed = _pack_tables(tables)
        acc = _sc_edge()(packed[0], packed[1], src, dst, zeros)[:, :NN, :]
        h1, st = _mlp1(acc, xin, W1, b1.reshape(1, HH))
        y, tnext = _mlp2(h1, st, g.reshape(1, HH), be.reshape(1, HH),
                         W2, b2.reshape(1, DD))
        return y, tnext

    t = _prep(x)
    y0, t = layer(t, x, W1_0, b1_0, gamma_0, beta_0, W2_0, b2_0)
    y1, _ = layer(t, y0, W1_1, b1_1, gamma_1, beta_1, W2_1, b2_1)
    return y1
